# baseline (speedup 1.0000x reference)
import functools

import jax
import jax.numpy as jnp
from jax import lax
from jax.experimental import pallas as pl
from jax.experimental.pallas import tpu as pltpu
from jax.experimental.pallas import tpu_sc as plsc

NC = 2
NS = 16
NW = NC * NS

E = 320000
D = 128
EW = E // NW
C = 128
NFULL = EW // C
CT = EW - NFULL * C
NCHUNK = NFULL + 1


def _sc_kernel(z_hbm, eidx_hbm, out_hbm,
               sidx_v, didx_v,
               srows0, srows1, srows2, drows0, drows1, drows2,
               out0, out1, out2, tr_v,
               sem_g0, sem_g1, sem_g2, sem_o0, sem_o1, sem_o2):
    SR = (srows0, srows1, srows2)
    DR = (drows0, drows1, drows2)
    OV = (out0, out1, out2)
    SEMG = (sem_g0, sem_g1, sem_g2)
    SEMO = (sem_o0, sem_o1, sem_o2)

    wid = lax.axis_index("s") * NC + lax.axis_index("c")
    base = wid * EW
    col0 = lax.iota(jnp.int32, 16) * 16

    def gather(b, k, n=C):
        o = k * C
        pltpu.async_copy(z_hbm.at[sidx_v.at[pl.ds(o, n)]],
                         SR[b].at[pl.ds(0, n)], SEMG[b])
        pltpu.async_copy(z_hbm.at[didx_v.at[pl.ds(o, n)]],
                         DR[b].at[pl.ds(0, n)], SEMG[b])

    def wait_gather(b, k, n=C):
        o = k * C
        pltpu.make_async_copy(z_hbm.at[sidx_v.at[pl.ds(o, n)]],
                              SR[b].at[pl.ds(0, n)], SEMG[b]).wait()
        pltpu.make_async_copy(z_hbm.at[didx_v.at[pl.ds(o, n)]],
                              DR[b].at[pl.ds(0, n)], SEMG[b]).wait()

    def wait_out(b, k, n=C):
        off = base + k * C
        pltpu.make_async_copy(OV[b].at[pl.ds(0, n)],
                              out_hbm.at[pl.ds(off, n)], SEMO[b]).wait()

    def compute(b, k, n=C):
        off = base + k * C
        srows_v = SR[b]
        drows_v = DR[b]
        out_v = OV[b]

        def group_body(g, _):
            e0 = g * 16
            for e in range(16):
                acc = (srows_v[e0 + e, pl.ds(0, 16)]
                       * drows_v[e0 + e, pl.ds(0, 16)])
                for j in range(1, D // 16):
                    acc = acc + (srows_v[e0 + e, pl.ds(j * 16, 16)]
                                 * drows_v[e0 + e, pl.ds(j * 16, 16)])
                tr_v[pl.ds(e * 16, 16)] = acc
            res = plsc.load_gather(tr_v, [col0])
            for j in range(1, 16):
                res = res + plsc.load_gather(tr_v, [col0 + j])
            out_v[pl.ds(e0, 16)] = res
            return 0

        lax.fori_loop(0, n // 16, group_body, 0)
        pltpu.async_copy(out_v.at[pl.ds(0, n)],
                         out_hbm.at[pl.ds(off, n)], SEMO[b])

    def step(k, b):
        gather((b + 2) % 3, k + 2)
        wait_gather(b, k)

        @pl.when(k >= 3)
        def _():
            wait_out(b, k - 3)

        compute(b, k)

    pltpu.sync_copy(eidx_hbm.at[pl.ds(base, EW)], sidx_v)
    pltpu.sync_copy(eidx_hbm.at[pl.ds(E + base, EW)], didx_v)
    gather(0, 0)
    gather(1, 1)

    def triple_body(i, _):
        step(3 * i, 0)
        step(3 * i + 1, 1)
        step(3 * i + 2, 2)
        return 0

    lax.fori_loop(0, (NFULL - 3) // 3, triple_body, 0)

    for k in range(3 * ((NFULL - 3) // 3), NCHUNK):
        b = k % 3
        if k + 2 < NCHUNK:
            gather((b + 2) % 3, k + 2, C if k + 2 < NFULL else CT)
        wait_gather(b, k, C if k < NFULL else CT)
        if k >= 3:
            wait_out(b, k - 3)
        compute(b, k, C if k < NFULL else CT)
    for k in range(max(NCHUNK - 3, 0), NCHUNK):
        wait_out(k % 3, k, C if k < NFULL else CT)


@jax.jit
def _run(z, eidx):
    mesh = plsc.VectorSubcoreMesh(core_axis_name="c", subcore_axis_name="s")
    return pl.kernel(
        _sc_kernel,
        out_type=jax.ShapeDtypeStruct((E,), jnp.float32),
        mesh=mesh,
        compiler_params=pltpu.CompilerParams(needs_layout_passes=False),
        scratch_types=[
            pltpu.VMEM((EW,), jnp.int32),
            pltpu.VMEM((EW,), jnp.int32),
            pltpu.VMEM((C, D), jnp.float32),
            pltpu.VMEM((C, D), jnp.float32),
            pltpu.VMEM((C, D), jnp.float32),
            pltpu.VMEM((C, D), jnp.float32),
            pltpu.VMEM((C, D), jnp.float32),
            pltpu.VMEM((C, D), jnp.float32),
            pltpu.VMEM((C,), jnp.float32),
            pltpu.VMEM((C,), jnp.float32),
            pltpu.VMEM((C,), jnp.float32),
            pltpu.VMEM((256,), jnp.float32),
            pltpu.SemaphoreType.DMA,
            pltpu.SemaphoreType.DMA,
            pltpu.SemaphoreType.DMA,
            pltpu.SemaphoreType.DMA,
            pltpu.SemaphoreType.DMA,
            pltpu.SemaphoreType.DMA,
        ],
    )(z, eidx)


def kernel(z, edge_index):
    return _run(z, edge_index.astype(jnp.int32).reshape(-1))


# speedup vs baseline: 1.0183x; 1.0183x over previous
import functools

import jax
import jax.numpy as jnp
from jax import lax
from jax.experimental import pallas as pl
from jax.experimental.pallas import tpu as pltpu
from jax.experimental.pallas import tpu_sc as plsc

NC = 2
NS = 16
NW = NC * NS

E = 320000
D = 128
EW = E // NW
C = 80
NCHUNK = EW // C


def _sc_kernel(z_hbm, eidx_hbm, out_hbm,
               sidx_v, didx_v,
               srows0, srows1, srows2, drows0, drows1, drows2,
               out0, out1, out2, tr_v,
               sem_g0, sem_g1, sem_g2, sem_o0, sem_o1, sem_o2):
    SR = (srows0, srows1, srows2)
    DR = (drows0, drows1, drows2)
    OV = (out0, out1, out2)
    SEMG = (sem_g0, sem_g1, sem_g2)
    SEMO = (sem_o0, sem_o1, sem_o2)

    wid = lax.axis_index("s") * NC + lax.axis_index("c")
    base = wid * EW
    col0 = lax.iota(jnp.int32, 16) * 16

    def gather(b, k):
        o = k * C
        pltpu.async_copy(z_hbm.at[sidx_v.at[pl.ds(o, C)]], SR[b], SEMG[b])
        pltpu.async_copy(z_hbm.at[didx_v.at[pl.ds(o, C)]], DR[b], SEMG[b])

    def wait_gather(b, k):
        o = k * C
        pltpu.make_async_copy(
            z_hbm.at[sidx_v.at[pl.ds(o, C)]], SR[b], SEMG[b]).wait()
        pltpu.make_async_copy(
            z_hbm.at[didx_v.at[pl.ds(o, C)]], DR[b], SEMG[b]).wait()

    def wait_out(b, k):
        off = base + k * C
        pltpu.make_async_copy(OV[b], out_hbm.at[pl.ds(off, C)], SEMO[b]).wait()

    def compute(b, k):
        off = base + k * C
        srows_v = SR[b]
        drows_v = DR[b]
        out_v = OV[b]

        def group_body(g, _):
            e0 = g * 16
            for e in range(16):
                acc = (srows_v[e0 + e, pl.ds(0, 16)]
                       * drows_v[e0 + e, pl.ds(0, 16)])
                for j in range(1, D // 16):
                    acc = acc + (srows_v[e0 + e, pl.ds(j * 16, 16)]
                                 * drows_v[e0 + e, pl.ds(j * 16, 16)])
                tr_v[pl.ds(e * 16, 16)] = acc
            res = plsc.load_gather(tr_v, [col0])
            for j in range(1, 16):
                res = res + plsc.load_gather(tr_v, [col0 + j])
            out_v[pl.ds(e0, 16)] = res
            return 0

        lax.fori_loop(0, C // 16, group_body, 0)
        pltpu.async_copy(out_v, out_hbm.at[pl.ds(off, C)], SEMO[b])

    def step(k, b):
        gather((b + 2) % 3, k + 2)
        wait_gather(b, k)

        @pl.when(k >= 3)
        def _():
            wait_out(b, k - 3)

        compute(b, k)

    pltpu.sync_copy(eidx_hbm.at[pl.ds(base, EW)], sidx_v)
    pltpu.sync_copy(eidx_hbm.at[pl.ds(E + base, EW)], didx_v)
    gather(0, 0)
    gather(1, 1)

    def triple_body(i, _):
        step(3 * i, 0)
        step(3 * i + 1, 1)
        step(3 * i + 2, 2)
        return 0

    lax.fori_loop(0, (NCHUNK - 2) // 3, triple_body, 0)

    for k in range(3 * ((NCHUNK - 2) // 3), NCHUNK):
        b = k % 3
        if k + 2 < NCHUNK:
            gather((b + 2) % 3, k + 2)
        wait_gather(b, k)
        if k >= 3:
            wait_out(b, k - 3)
        compute(b, k)
    for k in range(max(NCHUNK - 3, 0), NCHUNK):
        wait_out(k % 3, k)


@jax.jit
def _run(z, eidx):
    mesh = plsc.VectorSubcoreMesh(core_axis_name="c", subcore_axis_name="s")
    return pl.kernel(
        _sc_kernel,
        out_type=jax.ShapeDtypeStruct((E,), jnp.float32),
        mesh=mesh,
        compiler_params=pltpu.CompilerParams(needs_layout_passes=False),
        scratch_types=[
            pltpu.VMEM((EW,), jnp.int32),
            pltpu.VMEM((EW,), jnp.int32),
            pltpu.VMEM((C, D), jnp.float32),
            pltpu.VMEM((C, D), jnp.float32),
            pltpu.VMEM((C, D), jnp.float32),
            pltpu.VMEM((C, D), jnp.float32),
            pltpu.VMEM((C, D), jnp.float32),
            pltpu.VMEM((C, D), jnp.float32),
            pltpu.VMEM((C,), jnp.float32),
            pltpu.VMEM((C,), jnp.float32),
            pltpu.VMEM((C,), jnp.float32),
            pltpu.VMEM((256,), jnp.float32),
            pltpu.SemaphoreType.DMA,
            pltpu.SemaphoreType.DMA,
            pltpu.SemaphoreType.DMA,
            pltpu.SemaphoreType.DMA,
            pltpu.SemaphoreType.DMA,
            pltpu.SemaphoreType.DMA,
        ],
    )(z, eidx)


def kernel(z, edge_index):
    return _run(z, edge_index.astype(jnp.int32).reshape(-1))
